# baseline (speedup 1.0000x reference)
import functools

import jax
import jax.numpy as jnp
from jax import lax
from jax.experimental import pallas as pl
from jax.experimental.pallas import tpu as pltpu
from jax.experimental.pallas import tpu_sc as plsc

HIDDEN = 2048
INTER = 512
INTER_SHARED = 2048
NUM_EXPERTS = 64
TOP_K = 8
TOKENS = 32
NEG_INF = -1e30

J_SHARED = 16
CHUNK_SHARED = INTER_SHARED // J_SHARED

NLANES = 16
NVREG = NUM_EXPERTS // NLANES


def _logits_kernel(x_ref, rw_ref, out_ref):
    out_ref[:] = jnp.dot(x_ref[:], rw_ref[:],
                         preferred_element_type=jnp.float32)


def _shuffle(v, idx):
    return v.at[idx].get(mode="promise_in_bounds")


def _sc_routing_body(logits_hbm, out_hbm, buf, obuf):
    t = lax.axis_index("s") * 2 + lax.axis_index("c")
    pltpu.sync_copy(logits_hbm.at[t], buf)

    orig = [buf[pl.ds(j * NLANES, NLANES)] for j in range(NVREG)]
    vals = list(orig)
    sel = [jnp.zeros((NLANES,), jnp.bool_) for _ in range(NVREG)]
    iota = lax.iota(jnp.int32, NLANES)

    mtop = None
    for k in range(TOP_K):
        big = jnp.maximum(jnp.maximum(vals[0], vals[1]),
                          jnp.maximum(vals[2], vals[3]))
        for sh in (8, 4, 2, 1):
            big = jnp.maximum(big, _shuffle(big, jnp.bitwise_xor(iota, sh)))
        if k == 0:
            mtop = big
        cand = jnp.full((NLANES,), 127, jnp.int32)
        for j in range(NVREG):
            cand = jnp.minimum(
                cand, jnp.where(vals[j] == big, iota + 16 * j, 127))
        for sh in (8, 4, 2, 1):
            cand = jnp.minimum(cand, _shuffle(cand, jnp.bitwise_xor(iota, sh)))
        for j in range(NVREG):
            pick = (iota + 16 * j) == cand
            sel[j] = jnp.logical_or(sel[j], pick)
            vals[j] = jnp.where(pick, NEG_INF, vals[j])

    exps = [jnp.where(sel[j], jnp.exp(orig[j] - mtop), 0.0)
            for j in range(NVREG)]
    total = (exps[0] + exps[1]) + (exps[2] + exps[3])
    for sh in (8, 4, 2, 1):
        total = total + _shuffle(total, jnp.bitwise_xor(iota, sh))
    inv = 1.0 / total
    for j in range(NVREG):
        obuf[pl.ds(j * NLANES, NLANES)] = exps[j] * inv
    pltpu.sync_copy(obuf, out_hbm.at[t])


def _sc_routing(logits):
    mesh = plsc.VectorSubcoreMesh(core_axis_name="c", subcore_axis_name="s")
    f = functools.partial(
        pl.kernel,
        mesh=mesh,
        out_type=jax.ShapeDtypeStruct((TOKENS, NUM_EXPERTS), jnp.float32),
        scratch_types=[
            pltpu.VMEM((NUM_EXPERTS,), jnp.float32),
            pltpu.VMEM((NUM_EXPERTS,), jnp.float32),
        ],
    )(_sc_routing_body)
    return f(logits)


def _fused_kernel(x_ref, rt_ref, sgw_ref, gw_ref, iw_ref, sow_ref,
                  gu_ref, ow_ref, out_ref, sacc_ref):
    e = pl.program_id(0)
    x = x_ref[:]

    def _shared_chunk():
        g = jax.nn.silu(jnp.dot(x, gw_ref[:],
                                preferred_element_type=jnp.float32))
        i = jnp.dot(x, iw_ref[:], preferred_element_type=jnp.float32)
        return jnp.dot(g * i, sow_ref[:], preferred_element_type=jnp.float32)

    @pl.when(e == 0)
    def _init():
        sacc_ref[:] = _shared_chunk()

    @pl.when(jnp.logical_and(e > 0, e < J_SHARED))
    def _shared_acc():
        sacc_ref[:] += _shared_chunk()

    xw = jnp.dot(x, gu_ref[0], preferred_element_type=jnp.float32)
    gate = xw[:, :INTER]
    up = xw[:, INTER:]
    h = up * jax.nn.silu(gate)
    iota = jax.lax.broadcasted_iota(jnp.int32, rt_ref.shape, 1)
    w = jnp.sum(jnp.where(iota == e, rt_ref[:], 0.0), axis=-1, keepdims=True)
    contrib = jnp.dot(h * w, ow_ref[0], preferred_element_type=jnp.float32)

    @pl.when(e == 0)
    def _out_init():
        out_ref[:] = contrib

    @pl.when(e > 0)
    def _out_acc():
        out_ref[:] += contrib

    @pl.when(e == NUM_EXPERTS - 1)
    def _fin():
        sg = jax.nn.sigmoid(
            jnp.dot(x, sgw_ref[:], preferred_element_type=jnp.float32))
        out_ref[:] += sg * sacc_ref[:]


def _moe(x, router_w, expert_gate_up, expert_out_w, shared_gate_w,
         shared_inter_w, shared_out_w, shared_expert_gate_w):
    T = x.shape[0]
    logits = pl.pallas_call(
        _logits_kernel,
        out_shape=jax.ShapeDtypeStruct((T, NUM_EXPERTS), jnp.float32),
    )(x, router_w)

    routing = _sc_routing(logits)

    jcap = J_SHARED - 1
    out = pl.pallas_call(
        _fused_kernel,
        grid=(NUM_EXPERTS,),
        in_specs=[
            pl.BlockSpec((T, HIDDEN), lambda e: (0, 0)),
            pl.BlockSpec((T, NUM_EXPERTS), lambda e: (0, 0)),
            pl.BlockSpec((HIDDEN, 1), lambda e: (0, 0)),
            pl.BlockSpec((HIDDEN, CHUNK_SHARED),
                         lambda e: (0, jnp.minimum(e, jcap))),
            pl.BlockSpec((HIDDEN, CHUNK_SHARED),
                         lambda e: (0, jnp.minimum(e, jcap))),
            pl.BlockSpec((CHUNK_SHARED, HIDDEN),
                         lambda e: (jnp.minimum(e, jcap), 0)),
            pl.BlockSpec((1, HIDDEN, 2 * INTER), lambda e: (e, 0, 0)),
            pl.BlockSpec((1, INTER, HIDDEN), lambda e: (e, 0, 0)),
        ],
        out_specs=pl.BlockSpec((T, HIDDEN), lambda e: (0, 0)),
        out_shape=jax.ShapeDtypeStruct((T, HIDDEN), jnp.float32),
        scratch_shapes=[
            pltpu.VMEM((T, HIDDEN), jnp.float32),
        ],
        compiler_params=pltpu.CompilerParams(
            dimension_semantics=("arbitrary",)),
    )(x, routing, shared_expert_gate_w, shared_gate_w, shared_inter_w,
      shared_out_w, expert_gate_up, expert_out_w)
    return out


def kernel(hidden_states, router_w, expert_gate_up, expert_out_w,
           shared_gate_w, shared_inter_w, shared_out_w, shared_expert_gate_w):
    b, s, h = hidden_states.shape
    x = hidden_states.reshape(-1, h)
    out = _moe(x, router_w, expert_gate_up, expert_out_w, shared_gate_w,
               shared_inter_w, shared_out_w, shared_expert_gate_w)
    return out.reshape(b, s, h)
